# baseline (speedup 1.0000x reference)
import jax
import jax.numpy as jnp
from jax import lax
from jax.experimental import pallas as pl
from jax.experimental.pallas import tpu as pltpu
from jax.experimental.pallas import tpu_sc as plsc

N = 10000
E = 320000
R = 8
NB = 4

NUM_CORES = 2
NUM_SUBCORES = 16
NUM_WORKERS = NUM_CORES * NUM_SUBCORES
CHUNK = 128
CPW = 80
TOTAL_CHUNKS = NUM_WORKERS * CPW
E_PAD = TOTAL_CHUNKS * CHUNK
NBUF_FOR = {128: 2, 64: 2, 16: 8}

E_ROWS = E // CHUNK

BN = 1000
N_BLOCKS = N // BN

N_PAD = 10240
ROWS_PER_TILE = N_PAD // NUM_SUBCORES


def _meta_body(src_ref, et_ref, dst_ref, norm_ref, idx_ref, dsto_ref, mnorm_ref):
    lane = jax.lax.broadcasted_iota(jnp.int32, (TOTAL_CHUNKS - E_ROWS, CHUNK), 1)
    idx_ref[:E_ROWS] = src_ref[...] * R + et_ref[...]
    idx_ref[E_ROWS:] = lane
    dsto_ref[:E_ROWS] = dst_ref[...]
    dsto_ref[E_ROWS:] = lane + N
    mnorm_ref[:E_ROWS] = norm_ref[...]
    mnorm_ref[E_ROWS:] = jnp.zeros((TOTAL_CHUNKS - E_ROWS, CHUNK), jnp.float32)


def _compute_meta(src, et, dst, norm):
    f = pl.pallas_call(
        _meta_body,
        out_shape=[
            jax.ShapeDtypeStruct((TOTAL_CHUNKS, CHUNK), jnp.int32),
            jax.ShapeDtypeStruct((TOTAL_CHUNKS, CHUNK), jnp.int32),
            jax.ShapeDtypeStruct((TOTAL_CHUNKS, CHUNK), jnp.float32),
        ],
    )
    return f(src.reshape(E_ROWS, CHUNK), et.reshape(E_ROWS, CHUNK),
             dst.reshape(E_ROWS, CHUNK), norm.reshape(E_ROWS, CHUNK))


def _dense_first_body(h_ref, wb_ref, comp_ref, wl_ref, b_ref, hw_ref, loop_ref):
    h = h_ref[...]
    for r in range(R):
        w = comp_ref[r, 0] * wb_ref[0]
        for b in range(1, NB):
            w = w + comp_ref[r, b] * wb_ref[b]
        d = w.shape[1]
        hw_ref[:, r * d:(r + 1) * d] = jnp.dot(h, w, preferred_element_type=jnp.float32)
    loop_ref[...] = jnp.dot(h, wl_ref[...], preferred_element_type=jnp.float32) + b_ref[...]


def _dense_next_body(agg_ref, lp_ref, wb_ref, comp_ref, wl_ref, b_ref, hw_ref, loop_ref):
    x = agg_ref[0] + agg_ref[1] + lp_ref[...]
    h = jnp.where(x > 0, x, 0.01 * x)
    for r in range(R):
        w = comp_ref[r, 0] * wb_ref[0]
        for b in range(1, NB):
            w = w + comp_ref[r, b] * wb_ref[b]
        d = w.shape[1]
        hw_ref[:, r * d:(r + 1) * d] = jnp.dot(h, w, preferred_element_type=jnp.float32)
    loop_ref[...] = jnp.dot(h, wl_ref[...], preferred_element_type=jnp.float32) + b_ref[...]


def _dense_first(h, wb, comp, wl, b):
    din, d = wb.shape[1], wb.shape[2]
    f = pl.pallas_call(
        _dense_first_body,
        grid=(N_BLOCKS,),
        in_specs=[
            pl.BlockSpec((BN, din), lambda i: (i, 0)),
            pl.BlockSpec((NB, din, d), lambda i: (0, 0, 0)),
            pl.BlockSpec(memory_space=pltpu.SMEM),
            pl.BlockSpec((din, d), lambda i: (0, 0)),
            pl.BlockSpec((1, d), lambda i: (0, 0)),
        ],
        out_specs=[
            pl.BlockSpec((BN, R * d), lambda i: (i, 0)),
            pl.BlockSpec((BN, d), lambda i: (i, 0)),
        ],
        out_shape=[
            jax.ShapeDtypeStruct((N, R * d), jnp.float32),
            jax.ShapeDtypeStruct((N, d), jnp.float32),
        ],
    )
    return f(h, wb, comp, wl, b.reshape(1, d))


def _dense_next(agg, lp, wb, comp, wl, b):
    din, d = wb.shape[1], wb.shape[2]
    f = pl.pallas_call(
        _dense_next_body,
        grid=(N_BLOCKS,),
        in_specs=[
            pl.BlockSpec((2, BN, din), lambda i: (0, i, 0)),
            pl.BlockSpec((BN, din), lambda i: (i, 0)),
            pl.BlockSpec((NB, din, d), lambda i: (0, 0, 0)),
            pl.BlockSpec(memory_space=pltpu.SMEM),
            pl.BlockSpec((din, d), lambda i: (0, 0)),
            pl.BlockSpec((1, d), lambda i: (0, 0)),
        ],
        out_specs=[
            pl.BlockSpec((BN, R * d), lambda i: (i, 0)),
            pl.BlockSpec((BN, d), lambda i: (i, 0)),
        ],
        out_shape=[
            jax.ShapeDtypeStruct((N, R * d), jnp.float32),
            jax.ShapeDtypeStruct((N, d), jnp.float32),
        ],
    )
    return f(agg, lp, wb, comp, wl, b.reshape(1, d))


def _make_sc_scatter(d):
    mesh = plsc.VectorSubcoreMesh(core_axis_name="c", subcore_axis_name="s")
    nbuf = NBUF_FOR[d]

    def body(hw_hbm, idx_hbm, dst_hbm, mnorm_hbm, out_hbm, *scratch):
        idxv = list(scratch[:nbuf])
        dstv = list(scratch[nbuf:2 * nbuf])
        normv = list(scratch[2 * nbuf:3 * nbuf])
        rows_bufs = list(scratch[3 * nbuf:4 * nbuf])
        agg_sh = scratch[4 * nbuf]
        rest = list(scratch[4 * nbuf + 1:])
        msem = rest[0]
        gsems = rest[1:1 + nbuf]
        ssems = rest[1 + nbuf:1 + 2 * nbuf]

        c = lax.axis_index("c")
        s = lax.axis_index("s")
        wid = c * NUM_SUBCORES + s
        base_w = wid * CPW

        def zrow(i, carry):
            for jj in range(d // 16):
                rows_bufs[0][i, pl.ds(jj * 16, 16)] = jnp.zeros((16,), jnp.float32)
            return carry
        lax.fori_loop(0, CHUNK, zrow, 0)
        for k in range(ROWS_PER_TILE // CHUNK):
            pltpu.sync_copy(rows_bufs[0],
                            agg_sh.at[pl.ds(s * ROWS_PER_TILE + k * CHUNK, CHUNK)])
        plsc.subcore_barrier()

        def outer(j, carry):
            base = base_w + j * nbuf
            mds = []
            for b in range(nbuf):
                mds.append(pltpu.async_copy(idx_hbm.at[base + b], idxv[b], msem))
                mds.append(pltpu.async_copy(dst_hbm.at[base + b], dstv[b], msem))
                mds.append(pltpu.async_copy(mnorm_hbm.at[base + b], normv[b], msem))
            for md in mds:
                md.wait()
            gds = []
            for b in range(nbuf):
                gds.append(pltpu.async_copy(
                    hw_hbm.at[idxv[b]], rows_bufs[b], gsems[b]))
            sds = []
            for b in range(nbuf):
                gds[b].wait()
                rb = rows_bufs[b]
                nb = normv[b]

                def mul_group(g, carry2, _rb=rb, _nb=nb):
                    nv = _nb[pl.ds(g * 16, 16)]
                    for l in range(16):
                        sc = jnp.broadcast_to(nv[l], (16,))
                        e = g * 16 + l
                        for jj in range(d // 16):
                            _rb[e, pl.ds(jj * 16, 16)] = _rb[e, pl.ds(jj * 16, 16)] * sc
                    return carry2
                lax.fori_loop(0, CHUNK // 16, mul_group, 0)
                sds.append(pltpu.async_copy(
                    rb, agg_sh.at[dstv[b]], ssems[b], add=True))
            for sd in sds:
                sd.wait()
            return carry

        lax.fori_loop(0, CPW // nbuf, outer, 0)
        plsc.subcore_barrier()
        pltpu.sync_copy(agg_sh.at[pl.ds(s * ROWS_PER_TILE, ROWS_PER_TILE)],
                        out_hbm.at[c, pl.ds(s * ROWS_PER_TILE, ROWS_PER_TILE)])

    return pl.kernel(
        body,
        out_type=jax.ShapeDtypeStruct((NUM_CORES, N_PAD, d), jnp.float32),
        mesh=mesh,
        compiler_params=pltpu.CompilerParams(use_tc_tiling_on_sc=(d == 128)),
        scratch_types=(
            [pltpu.VMEM((CHUNK,), jnp.int32)] * nbuf
            + [pltpu.VMEM((CHUNK,), jnp.int32)] * nbuf
            + [pltpu.VMEM((CHUNK,), jnp.float32)] * nbuf
            + [pltpu.VMEM((CHUNK, d), jnp.float32)] * nbuf
            + [pltpu.VMEM_SHARED((N_PAD, d), jnp.float32)]
            + [pltpu.SemaphoreType.DMA] * (1 + 2 * nbuf)
        ),
    )


_sc_scatter_cache = {}


def _sc_scatter(d, hw_flat, idx2d, dst2d, mnorm):
    if d not in _sc_scatter_cache:
        _sc_scatter_cache[d] = _make_sc_scatter(d)
    return _sc_scatter_cache[d](hw_flat, idx2d, dst2d, mnorm)


def _readout_body(a_ref, l_ref, o_ref):
    o_ref[...] = jnp.tanh(a_ref[0] + a_ref[1] + l_ref[...])


def _readout(a, l):
    f = pl.pallas_call(
        _readout_body,
        out_shape=jax.ShapeDtypeStruct(l.shape, jnp.float32),
    )
    return f(a, l)


def kernel(features, etypes, edge_index, norm,
           Wb1, comp1, Wloop1, b1,
           Wb2, comp2, Wloop2, b2,
           Wb3, comp3, Wloop3, b3):
    src = edge_index[0].astype(jnp.int32)
    dst = edge_index[1].astype(jnp.int32)
    et = etypes.astype(jnp.int32)

    idx2d, dst2d, mnorm = _compute_meta(src, et, dst, norm.reshape(E))

    d3 = 16
    Wb3p = jnp.pad(Wb3, ((0, 0), (0, 0), (0, d3 - Wb3.shape[2])))
    Wloop3p = jnp.pad(Wloop3, ((0, 0), (0, d3 - Wloop3.shape[1])))
    b3p = jnp.pad(b3, (0, d3 - b3.shape[0]))

    hw1, loop1 = _dense_first(features, Wb1, comp1, Wloop1, b1)
    agg1 = _sc_scatter(128, hw1.reshape(N * R, 128), idx2d, dst2d, mnorm)

    hw2, loop2 = _dense_next(agg1, loop1, Wb2, comp2, Wloop2, b2)
    agg2 = _sc_scatter(64, hw2.reshape(N * R, 64), idx2d, dst2d, mnorm)

    hw3, loop3 = _dense_next(agg2, loop2, Wb3p, comp3, Wloop3p, b3p)
    agg3 = _sc_scatter(d3, hw3.reshape(N * R, d3), idx2d, dst2d, mnorm)

    offsets = jnp.arange(16) * (N // 16)
    a = agg3[:, offsets, :]
    l = loop3[offsets, :]
    out = _readout(a, l)
    return out[:, :3]


# speedup vs baseline: 1.1289x; 1.1289x over previous
import jax
import jax.numpy as jnp
from jax import lax
from jax.experimental import pallas as pl
from jax.experimental.pallas import tpu as pltpu
from jax.experimental.pallas import tpu_sc as plsc

N = 10000
E = 320000
R = 8
NB = 4

NUM_CORES = 2
NUM_SUBCORES = 16
NUM_WORKERS = NUM_CORES * NUM_SUBCORES
CHUNK = 128
CPW = 80
TOTAL_CHUNKS = NUM_WORKERS * CPW
E_PAD = TOTAL_CHUNKS * CHUNK
NBUF_FOR = {128: 2, 64: 4, 16: 8}

E_ROWS = E // CHUNK

BN = 1000
N_BLOCKS = N // BN

N_PAD = 10240
ROWS_PER_TILE = N_PAD // NUM_SUBCORES


def _meta_body(src_ref, et_ref, dst_ref, norm_ref, idx_ref, dsto_ref, mnorm_ref):
    lane = jax.lax.broadcasted_iota(jnp.int32, (TOTAL_CHUNKS - E_ROWS, CHUNK), 1)
    idx_ref[:E_ROWS] = src_ref[...] * R + et_ref[...]
    idx_ref[E_ROWS:] = lane
    dsto_ref[:E_ROWS] = dst_ref[...]
    dsto_ref[E_ROWS:] = lane + N
    mnorm_ref[:E_ROWS] = norm_ref[...]
    mnorm_ref[E_ROWS:] = jnp.zeros((TOTAL_CHUNKS - E_ROWS, CHUNK), jnp.float32)


def _compute_meta(src, et, dst, norm):
    f = pl.pallas_call(
        _meta_body,
        out_shape=[
            jax.ShapeDtypeStruct((TOTAL_CHUNKS, CHUNK), jnp.int32),
            jax.ShapeDtypeStruct((TOTAL_CHUNKS, CHUNK), jnp.int32),
            jax.ShapeDtypeStruct((TOTAL_CHUNKS, CHUNK), jnp.float32),
        ],
    )
    return f(src.reshape(E_ROWS, CHUNK), et.reshape(E_ROWS, CHUNK),
             dst.reshape(E_ROWS, CHUNK), norm.reshape(E_ROWS, CHUNK))


def _dense_first_body(h_ref, wb_ref, comp_ref, wl_ref, b_ref, hw_ref, loop_ref):
    h = h_ref[...]
    for r in range(R):
        w = comp_ref[r, 0] * wb_ref[0]
        for b in range(1, NB):
            w = w + comp_ref[r, b] * wb_ref[b]
        d = w.shape[1]
        hw_ref[:, r * d:(r + 1) * d] = jnp.dot(h, w, preferred_element_type=jnp.float32)
    loop_ref[...] = jnp.dot(h, wl_ref[...], preferred_element_type=jnp.float32) + b_ref[...]


def _dense_next_body(agg_ref, lp_ref, wb_ref, comp_ref, wl_ref, b_ref, hw_ref, loop_ref):
    x = agg_ref[0] + agg_ref[1] + lp_ref[...]
    h = jnp.where(x > 0, x, 0.01 * x)
    for r in range(R):
        w = comp_ref[r, 0] * wb_ref[0]
        for b in range(1, NB):
            w = w + comp_ref[r, b] * wb_ref[b]
        d = w.shape[1]
        hw_ref[:, r * d:(r + 1) * d] = jnp.dot(h, w, preferred_element_type=jnp.float32)
    loop_ref[...] = jnp.dot(h, wl_ref[...], preferred_element_type=jnp.float32) + b_ref[...]


def _dense_first(h, wb, comp, wl, b):
    din, d = wb.shape[1], wb.shape[2]
    f = pl.pallas_call(
        _dense_first_body,
        grid=(N_BLOCKS,),
        in_specs=[
            pl.BlockSpec((BN, din), lambda i: (i, 0)),
            pl.BlockSpec((NB, din, d), lambda i: (0, 0, 0)),
            pl.BlockSpec(memory_space=pltpu.SMEM),
            pl.BlockSpec((din, d), lambda i: (0, 0)),
            pl.BlockSpec((1, d), lambda i: (0, 0)),
        ],
        out_specs=[
            pl.BlockSpec((BN, R * d), lambda i: (i, 0)),
            pl.BlockSpec((BN, d), lambda i: (i, 0)),
        ],
        out_shape=[
            jax.ShapeDtypeStruct((N, R * d), jnp.float32),
            jax.ShapeDtypeStruct((N, d), jnp.float32),
        ],
    )
    return f(h, wb, comp, wl, b.reshape(1, d))


def _dense_next(agg, lp, wb, comp, wl, b):
    din, d = wb.shape[1], wb.shape[2]
    f = pl.pallas_call(
        _dense_next_body,
        grid=(N_BLOCKS,),
        in_specs=[
            pl.BlockSpec((2, BN, din), lambda i: (0, i, 0)),
            pl.BlockSpec((BN, din), lambda i: (i, 0)),
            pl.BlockSpec((NB, din, d), lambda i: (0, 0, 0)),
            pl.BlockSpec(memory_space=pltpu.SMEM),
            pl.BlockSpec((din, d), lambda i: (0, 0)),
            pl.BlockSpec((1, d), lambda i: (0, 0)),
        ],
        out_specs=[
            pl.BlockSpec((BN, R * d), lambda i: (i, 0)),
            pl.BlockSpec((BN, d), lambda i: (i, 0)),
        ],
        out_shape=[
            jax.ShapeDtypeStruct((N, R * d), jnp.float32),
            jax.ShapeDtypeStruct((N, d), jnp.float32),
        ],
    )
    return f(agg, lp, wb, comp, wl, b.reshape(1, d))


def _make_sc_scatter(d):
    mesh = plsc.VectorSubcoreMesh(core_axis_name="c", subcore_axis_name="s")
    nbuf = NBUF_FOR[d]

    def body(hw_hbm, idx_hbm, dst_hbm, mnorm_hbm, out_hbm, *scratch):
        idxv = list(scratch[:nbuf])
        dstv = list(scratch[nbuf:2 * nbuf])
        normv = list(scratch[2 * nbuf:3 * nbuf])
        rows_bufs = list(scratch[3 * nbuf:4 * nbuf])
        agg_sh = scratch[4 * nbuf]
        rest = list(scratch[4 * nbuf + 1:])
        msem = rest[0]
        gsems = rest[1:1 + nbuf]
        ssems = rest[1 + nbuf:1 + 2 * nbuf]

        c = lax.axis_index("c")
        s = lax.axis_index("s")
        wid = c * NUM_SUBCORES + s
        base_w = wid * CPW

        def zrow(i, carry):
            for jj in range(d // 16):
                rows_bufs[0][i, pl.ds(jj * 16, 16)] = jnp.zeros((16,), jnp.float32)
            return carry
        lax.fori_loop(0, CHUNK, zrow, 0)
        for k in range(ROWS_PER_TILE // CHUNK):
            pltpu.sync_copy(rows_bufs[0],
                            agg_sh.at[pl.ds(s * ROWS_PER_TILE + k * CHUNK, CHUNK)])
        plsc.subcore_barrier()

        def outer(j, carry):
            base = base_w + j * nbuf
            mds = []
            for b in range(nbuf):
                mds.append(pltpu.async_copy(idx_hbm.at[base + b], idxv[b], msem))
                mds.append(pltpu.async_copy(dst_hbm.at[base + b], dstv[b], msem))
                mds.append(pltpu.async_copy(mnorm_hbm.at[base + b], normv[b], msem))
            for md in mds:
                md.wait()
            gds = []
            for b in range(nbuf):
                gds.append(pltpu.async_copy(
                    hw_hbm.at[idxv[b]], rows_bufs[b], gsems[b]))
            sds = []
            for b in range(nbuf):
                gds[b].wait()
                rb = rows_bufs[b]
                nb = normv[b]

                def mul_group(g, carry2, _rb=rb, _nb=nb):
                    nv = _nb[pl.ds(g * 16, 16)]
                    for l in range(16):
                        sc = jnp.broadcast_to(nv[l], (16,))
                        e = g * 16 + l
                        for jj in range(d // 16):
                            _rb[e, pl.ds(jj * 16, 16)] = _rb[e, pl.ds(jj * 16, 16)] * sc
                    return carry2
                lax.fori_loop(0, CHUNK // 16, mul_group, 0)
                sds.append(pltpu.async_copy(
                    rb, agg_sh.at[dstv[b]], ssems[b], add=True))
            for sd in sds:
                sd.wait()
            return carry

        lax.fori_loop(0, CPW // nbuf, outer, 0)
        plsc.subcore_barrier()
        pltpu.sync_copy(agg_sh.at[pl.ds(s * ROWS_PER_TILE, ROWS_PER_TILE)],
                        out_hbm.at[c, pl.ds(s * ROWS_PER_TILE, ROWS_PER_TILE)])

    return pl.kernel(
        body,
        out_type=jax.ShapeDtypeStruct((NUM_CORES, N_PAD, d), jnp.float32),
        mesh=mesh,
        compiler_params=pltpu.CompilerParams(use_tc_tiling_on_sc=False),
        scratch_types=(
            [pltpu.VMEM((CHUNK,), jnp.int32)] * nbuf
            + [pltpu.VMEM((CHUNK,), jnp.int32)] * nbuf
            + [pltpu.VMEM((CHUNK,), jnp.float32)] * nbuf
            + [pltpu.VMEM((CHUNK, d), jnp.float32)] * nbuf
            + [pltpu.VMEM_SHARED((N_PAD, d), jnp.float32)]
            + [pltpu.SemaphoreType.DMA] * (1 + 2 * nbuf)
        ),
    )


_sc_scatter_cache = {}


def _sc_scatter(d, hw_flat, idx2d, dst2d, mnorm):
    if d not in _sc_scatter_cache:
        _sc_scatter_cache[d] = _make_sc_scatter(d)
    return _sc_scatter_cache[d](hw_flat, idx2d, dst2d, mnorm)


def _readout_body(a_ref, l_ref, o_ref):
    for k in range(16):
        r = k * (N // 16)
        o_ref[k:k + 1, :] = jnp.tanh(
            a_ref[0, r:r + 1, :] + a_ref[1, r:r + 1, :] + l_ref[r:r + 1, :])


def _readout(agg3, loop3):
    f = pl.pallas_call(
        _readout_body,
        out_shape=jax.ShapeDtypeStruct((16, loop3.shape[1]), jnp.float32),
    )
    return f(agg3, loop3)


def kernel(features, etypes, edge_index, norm,
           Wb1, comp1, Wloop1, b1,
           Wb2, comp2, Wloop2, b2,
           Wb3, comp3, Wloop3, b3):
    src = edge_index[0].astype(jnp.int32)
    dst = edge_index[1].astype(jnp.int32)
    et = etypes.astype(jnp.int32)

    idx2d, dst2d, mnorm = _compute_meta(src, et, dst, norm.reshape(E))

    d3 = 16
    Wb3p = jnp.pad(Wb3, ((0, 0), (0, 0), (0, d3 - Wb3.shape[2])))
    Wloop3p = jnp.pad(Wloop3, ((0, 0), (0, d3 - Wloop3.shape[1])))
    b3p = jnp.pad(b3, (0, d3 - b3.shape[0]))

    hw1, loop1 = _dense_first(features, Wb1, comp1, Wloop1, b1)
    agg1 = _sc_scatter(128, hw1.reshape(N * R, 128), idx2d, dst2d, mnorm)

    hw2, loop2 = _dense_next(agg1, loop1, Wb2, comp2, Wloop2, b2)
    agg2 = _sc_scatter(64, hw2.reshape(N * R, 64), idx2d, dst2d, mnorm)

    hw3, loop3 = _dense_next(agg2, loop2, Wb3p, comp3, Wloop3p, b3p)
    agg3 = _sc_scatter(d3, hw3.reshape(N * R, d3), idx2d, dst2d, mnorm)

    out = _readout(agg3, loop3)
    return out[:, :3]


# speedup vs baseline: 1.1601x; 1.0277x over previous
import jax
import jax.numpy as jnp
from jax import lax
from jax.experimental import pallas as pl
from jax.experimental.pallas import tpu as pltpu
from jax.experimental.pallas import tpu_sc as plsc

N = 10000
E = 320000
R = 8
NB = 4

NUM_CORES = 2
NUM_SUBCORES = 16
NUM_WORKERS = NUM_CORES * NUM_SUBCORES
CHUNK = 128
CPW = 80
TOTAL_CHUNKS = NUM_WORKERS * CPW
E_PAD = TOTAL_CHUNKS * CHUNK
NBUF_FOR = {128: 2, 64: 8, 16: 8}

E_ROWS = E // CHUNK

BN = 1000
N_BLOCKS = N // BN

N_PAD = 10240
ROWS_PER_TILE = N_PAD // NUM_SUBCORES


def _meta_body(src_ref, et_ref, dst_ref, norm_ref, idx_ref, dsto_ref, mnorm_ref):
    lane = jax.lax.broadcasted_iota(jnp.int32, (TOTAL_CHUNKS - E_ROWS, CHUNK), 1)
    idx_ref[:E_ROWS] = src_ref[...] * R + et_ref[...]
    idx_ref[E_ROWS:] = lane
    dsto_ref[:E_ROWS] = dst_ref[...]
    dsto_ref[E_ROWS:] = lane + N
    mnorm_ref[:E_ROWS] = norm_ref[...]
    mnorm_ref[E_ROWS:] = jnp.zeros((TOTAL_CHUNKS - E_ROWS, CHUNK), jnp.float32)


def _compute_meta(src, et, dst, norm):
    f = pl.pallas_call(
        _meta_body,
        out_shape=[
            jax.ShapeDtypeStruct((TOTAL_CHUNKS, CHUNK), jnp.int32),
            jax.ShapeDtypeStruct((TOTAL_CHUNKS, CHUNK), jnp.int32),
            jax.ShapeDtypeStruct((TOTAL_CHUNKS, CHUNK), jnp.float32),
        ],
    )
    return f(src.reshape(E_ROWS, CHUNK), et.reshape(E_ROWS, CHUNK),
             dst.reshape(E_ROWS, CHUNK), norm.reshape(E_ROWS, CHUNK))


def _dense_first_body(h_ref, wb_ref, comp_ref, wl_ref, b_ref, hw_ref, loop_ref):
    h = h_ref[...]
    for r in range(R):
        w = comp_ref[r, 0] * wb_ref[0]
        for b in range(1, NB):
            w = w + comp_ref[r, b] * wb_ref[b]
        d = w.shape[1]
        hw_ref[:, r * d:(r + 1) * d] = jnp.dot(h, w, preferred_element_type=jnp.float32)
    loop_ref[...] = jnp.dot(h, wl_ref[...], preferred_element_type=jnp.float32) + b_ref[...]


def _dense_next_body(agg_ref, lp_ref, wb_ref, comp_ref, wl_ref, b_ref, hw_ref, loop_ref):
    x = agg_ref[0] + agg_ref[1] + lp_ref[...]
    h = jnp.where(x > 0, x, 0.01 * x)
    for r in range(R):
        w = comp_ref[r, 0] * wb_ref[0]
        for b in range(1, NB):
            w = w + comp_ref[r, b] * wb_ref[b]
        d = w.shape[1]
        hw_ref[:, r * d:(r + 1) * d] = jnp.dot(h, w, preferred_element_type=jnp.float32)
    loop_ref[...] = jnp.dot(h, wl_ref[...], preferred_element_type=jnp.float32) + b_ref[...]


def _dense_first(h, wb, comp, wl, b):
    din, d = wb.shape[1], wb.shape[2]
    f = pl.pallas_call(
        _dense_first_body,
        grid=(N_BLOCKS,),
        in_specs=[
            pl.BlockSpec((BN, din), lambda i: (i, 0)),
            pl.BlockSpec((NB, din, d), lambda i: (0, 0, 0)),
            pl.BlockSpec(memory_space=pltpu.SMEM),
            pl.BlockSpec((din, d), lambda i: (0, 0)),
            pl.BlockSpec((1, d), lambda i: (0, 0)),
        ],
        out_specs=[
            pl.BlockSpec((BN, R * d), lambda i: (i, 0)),
            pl.BlockSpec((BN, d), lambda i: (i, 0)),
        ],
        out_shape=[
            jax.ShapeDtypeStruct((N, R * d), jnp.float32),
            jax.ShapeDtypeStruct((N, d), jnp.float32),
        ],
    )
    return f(h, wb, comp, wl, b.reshape(1, d))


def _dense_next(agg, lp, wb, comp, wl, b):
    din, d = wb.shape[1], wb.shape[2]
    f = pl.pallas_call(
        _dense_next_body,
        grid=(N_BLOCKS,),
        in_specs=[
            pl.BlockSpec((2, BN, din), lambda i: (0, i, 0)),
            pl.BlockSpec((BN, din), lambda i: (i, 0)),
            pl.BlockSpec((NB, din, d), lambda i: (0, 0, 0)),
            pl.BlockSpec(memory_space=pltpu.SMEM),
            pl.BlockSpec((din, d), lambda i: (0, 0)),
            pl.BlockSpec((1, d), lambda i: (0, 0)),
        ],
        out_specs=[
            pl.BlockSpec((BN, R * d), lambda i: (i, 0)),
            pl.BlockSpec((BN, d), lambda i: (i, 0)),
        ],
        out_shape=[
            jax.ShapeDtypeStruct((N, R * d), jnp.float32),
            jax.ShapeDtypeStruct((N, d), jnp.float32),
        ],
    )
    return f(agg, lp, wb, comp, wl, b.reshape(1, d))


def _make_sc_scatter(d):
    mesh = plsc.VectorSubcoreMesh(core_axis_name="c", subcore_axis_name="s")
    nbuf = NBUF_FOR[d]

    def body(hw_hbm, idx_hbm, dst_hbm, mnorm_hbm, out_hbm, *scratch):
        idxv = list(scratch[:nbuf])
        dstv = list(scratch[nbuf:2 * nbuf])
        normv = list(scratch[2 * nbuf:3 * nbuf])
        rows_bufs = list(scratch[3 * nbuf:4 * nbuf])
        agg_sh = scratch[4 * nbuf]
        rest = list(scratch[4 * nbuf + 1:])
        msem = rest[0]
        gsems = rest[1:1 + nbuf]
        ssems = rest[1 + nbuf:1 + 2 * nbuf]

        c = lax.axis_index("c")
        s = lax.axis_index("s")
        wid = c * NUM_SUBCORES + s
        base_w = wid * CPW

        def zrow(i, carry):
            for jj in range(d // 16):
                rows_bufs[0][i, pl.ds(jj * 16, 16)] = jnp.zeros((16,), jnp.float32)
            return carry
        lax.fori_loop(0, CHUNK, zrow, 0)
        for k in range(ROWS_PER_TILE // CHUNK):
            pltpu.sync_copy(rows_bufs[0],
                            agg_sh.at[pl.ds(s * ROWS_PER_TILE + k * CHUNK, CHUNK)])
        plsc.subcore_barrier()

        def outer(j, carry):
            base = base_w + j * nbuf
            mds = []
            for b in range(nbuf):
                mds.append(pltpu.async_copy(idx_hbm.at[base + b], idxv[b], msem))
                mds.append(pltpu.async_copy(dst_hbm.at[base + b], dstv[b], msem))
                mds.append(pltpu.async_copy(mnorm_hbm.at[base + b], normv[b], msem))
            for md in mds:
                md.wait()
            gds = []
            for b in range(nbuf):
                gds.append(pltpu.async_copy(
                    hw_hbm.at[idxv[b]], rows_bufs[b], gsems[b]))
            sds = []
            for b in range(nbuf):
                gds[b].wait()
                rb = rows_bufs[b]
                nb = normv[b]

                def mul_group(g, carry2, _rb=rb, _nb=nb):
                    nv = _nb[pl.ds(g * 16, 16)]
                    for l in range(16):
                        sc = jnp.broadcast_to(nv[l], (16,))
                        e = g * 16 + l
                        for jj in range(d // 16):
                            _rb[e, pl.ds(jj * 16, 16)] = _rb[e, pl.ds(jj * 16, 16)] * sc
                    return carry2
                lax.fori_loop(0, CHUNK // 16, mul_group, 0)
                sds.append(pltpu.async_copy(
                    rb, agg_sh.at[dstv[b]], ssems[b], add=True))
            for sd in sds:
                sd.wait()
            return carry

        lax.fori_loop(0, CPW // nbuf, outer, 0)
        plsc.subcore_barrier()
        pltpu.sync_copy(agg_sh.at[pl.ds(s * ROWS_PER_TILE, ROWS_PER_TILE)],
                        out_hbm.at[c, pl.ds(s * ROWS_PER_TILE, ROWS_PER_TILE)])

    return pl.kernel(
        body,
        out_type=jax.ShapeDtypeStruct((NUM_CORES, N_PAD, d), jnp.float32),
        mesh=mesh,
        compiler_params=pltpu.CompilerParams(use_tc_tiling_on_sc=False),
        scratch_types=(
            [pltpu.VMEM((CHUNK,), jnp.int32)] * nbuf
            + [pltpu.VMEM((CHUNK,), jnp.int32)] * nbuf
            + [pltpu.VMEM((CHUNK,), jnp.float32)] * nbuf
            + [pltpu.VMEM((CHUNK, d), jnp.float32)] * nbuf
            + [pltpu.VMEM_SHARED((N_PAD, d), jnp.float32)]
            + [pltpu.SemaphoreType.DMA] * (1 + 2 * nbuf)
        ),
    )


_sc_scatter_cache = {}


def _sc_scatter(d, hw_flat, idx2d, dst2d, mnorm):
    if d not in _sc_scatter_cache:
        _sc_scatter_cache[d] = _make_sc_scatter(d)
    return _sc_scatter_cache[d](hw_flat, idx2d, dst2d, mnorm)


def _readout_body(a_ref, l_ref, o_ref):
    for k in range(16):
        r = k * (N // 16)
        o_ref[k:k + 1, :] = jnp.tanh(
            a_ref[0, r:r + 1, :] + a_ref[1, r:r + 1, :] + l_ref[r:r + 1, :])


def _readout(agg3, loop3):
    f = pl.pallas_call(
        _readout_body,
        out_shape=jax.ShapeDtypeStruct((16, loop3.shape[1]), jnp.float32),
    )
    return f(agg3, loop3)


def kernel(features, etypes, edge_index, norm,
           Wb1, comp1, Wloop1, b1,
           Wb2, comp2, Wloop2, b2,
           Wb3, comp3, Wloop3, b3):
    src = edge_index[0].astype(jnp.int32)
    dst = edge_index[1].astype(jnp.int32)
    et = etypes.astype(jnp.int32)

    idx2d, dst2d, mnorm = _compute_meta(src, et, dst, norm.reshape(E))

    d3 = 16
    Wb3p = jnp.pad(Wb3, ((0, 0), (0, 0), (0, d3 - Wb3.shape[2])))
    Wloop3p = jnp.pad(Wloop3, ((0, 0), (0, d3 - Wloop3.shape[1])))
    b3p = jnp.pad(b3, (0, d3 - b3.shape[0]))

    hw1, loop1 = _dense_first(features, Wb1, comp1, Wloop1, b1)
    agg1 = _sc_scatter(128, hw1.reshape(N * R, 128), idx2d, dst2d, mnorm)

    hw2, loop2 = _dense_next(agg1, loop1, Wb2, comp2, Wloop2, b2)
    agg2 = _sc_scatter(64, hw2.reshape(N * R, 64), idx2d, dst2d, mnorm)

    hw3, loop3 = _dense_next(agg2, loop2, Wb3p, comp3, Wloop3p, b3p)
    agg3 = _sc_scatter(d3, hw3.reshape(N * R, d3), idx2d, dst2d, mnorm)

    out = _readout(agg3, loop3)
    return out[:, :3]
